# Initial kernel scaffold; baseline (speedup 1.0000x reference)
import jax
import jax.numpy as jnp
from jax.experimental import pallas as pl


def kernel(hidden_states, attention_mask, self_attention_scores):
    raise NotImplementedError("write your pallas kernel here")


# speedup vs baseline: 4.8409x; 4.8409x over previous
import jax
import jax.numpy as jnp
from jax.experimental import pallas as pl
from jax.experimental.pallas import tpu as pltpu

T = 1024
D = 768


def _merge_body(src_ref, dst_ref, out_ref):
    src = src_ref[0]
    dst = dst_ref[0]
    sn = src / jnp.sqrt(jnp.sum(src * src, axis=1, keepdims=True))
    dn = dst / jnp.sqrt(jnp.sum(dst * dst, axis=1, keepdims=True))
    scores_t = jax.lax.dot_general(
        dn, sn, (((1,), (1,)), ((), ())), preferred_element_type=jnp.float32
    )
    m = jnp.max(scores_t, axis=0, keepdims=True)
    jj = jax.lax.broadcasted_iota(jnp.int32, (T, T), 0)
    idx_row = jnp.min(
        jnp.where(scores_t == m, jj, jnp.int32(2**30)), axis=0, keepdims=True
    )
    ii = jax.lax.broadcasted_iota(jnp.int32, (1, T), 1)
    idx_row = jnp.where(ii == 0, jnp.int32(T), idx_row)
    et = (jj == idx_row).astype(jnp.float32)
    sums = dst + jax.lax.dot_general(
        et, src, (((1,), (0,)), ((), ())), preferred_element_type=jnp.float32
    )
    counts = 1.0 + jnp.sum(et, axis=1, keepdims=True)
    out_ref[0] = sums / counts


def _bipartite_merge_tc(src, dst, interpret=False):
    B = src.shape[0]
    return pl.pallas_call(
        _merge_body,
        grid=(B,),
        in_specs=[
            pl.BlockSpec((1, T, D), lambda b: (b, 0, 0)),
            pl.BlockSpec((1, T, D), lambda b: (b, 0, 0)),
        ],
        out_specs=pl.BlockSpec((1, T, D), lambda b: (b, 0, 0)),
        out_shape=jax.ShapeDtypeStruct((B, T, D), jnp.float32),
        interpret=interpret,
    )(src, dst)


def kernel(hidden_states, attention_mask, self_attention_scores):
    B, L, Dd = hidden_states.shape
    assert L == 2 * T and Dd == D
    h = hidden_states.reshape(B, T, 2, D)
    src = h[:, :, 0, :]
    dst = h[:, :, 1, :]
    dst_m = _bipartite_merge_tc(src, dst)
    preserved = jnp.concatenate([hidden_states[:, :1, :], dst_m], axis=1)
    mask = jnp.zeros((B, 1, 1, T + 1), dtype=self_attention_scores.dtype)
    return preserved, mask
